# baseline (speedup 1.0000x reference)
import jax
import jax.numpy as jnp
from jax.experimental import pallas as pl
from jax.experimental.pallas import tpu as pltpu

_BLOCK_ROWS = 400


def _logits_kernel(feat_ref, b_feat_ref, w_feat_ref, w_lin_ref, b_lin_ref,
                   out_ref):
    w_eff = jax.lax.dot(
        w_feat_ref[:], w_lin_ref[:],
        precision=jax.lax.Precision.HIGHEST,
        preferred_element_type=jnp.float32,
    )
    x = feat_ref[:] + b_feat_ref[:]
    acc = jnp.sum(x * w_eff[:, 0][None, :], axis=1, keepdims=True)
    out_ref[:] = acc + b_lin_ref[:]


def kernel(feat, edge_index, b_feat, W_feat, W_n, b_n, W_lin, b_lin):
    del edge_index, W_n, b_n
    n, f = feat.shape
    grid = (n // _BLOCK_ROWS,)
    b_lin2d = b_lin.reshape(1, 1)
    return pl.pallas_call(
        _logits_kernel,
        grid=grid,
        in_specs=[
            pl.BlockSpec((_BLOCK_ROWS, f), lambda i: (i, 0)),
            pl.BlockSpec((1, f), lambda i: (0, 0)),
            pl.BlockSpec(W_feat.shape, lambda i: (0, 0)),
            pl.BlockSpec(W_lin.shape, lambda i: (0, 0)),
            pl.BlockSpec((1, 1), lambda i: (0, 0)),
        ],
        out_specs=pl.BlockSpec((_BLOCK_ROWS, 1), lambda i: (i, 0)),
        out_shape=jax.ShapeDtypeStruct((n, 1), jnp.float32),
        compiler_params=pltpu.CompilerParams(
            dimension_semantics=("parallel",),
        ),
    )(feat, b_feat, W_feat, W_lin, b_lin2d)


# speedup vs baseline: 1.8498x; 1.8498x over previous
import jax
import jax.numpy as jnp
from jax.experimental import pallas as pl
from jax.experimental.pallas import tpu as pltpu

_BLOCK_ROWS = 2000


def _logits_kernel(feat_ref, b_feat_ref, w_feat_ref, w_lin_ref, b_lin_ref,
                   out_ref):
    w_eff = jax.lax.dot(
        w_feat_ref[:], w_lin_ref[:],
        precision=jax.lax.Precision.HIGHEST,
        preferred_element_type=jnp.float32,
    )
    x = feat_ref[:] + b_feat_ref[:]
    acc = jnp.sum(x * w_eff[:, 0][None, :], axis=1, keepdims=True)
    out_ref[:] = acc + b_lin_ref[:]


def kernel(feat, edge_index, b_feat, W_feat, W_n, b_n, W_lin, b_lin):
    del edge_index, W_n, b_n
    n, f = feat.shape
    grid = (n // _BLOCK_ROWS,)
    b_lin2d = b_lin.reshape(1, 1)
    return pl.pallas_call(
        _logits_kernel,
        grid=grid,
        in_specs=[
            pl.BlockSpec((_BLOCK_ROWS, f), lambda i: (i, 0)),
            pl.BlockSpec((1, f), lambda i: (0, 0)),
            pl.BlockSpec(W_feat.shape, lambda i: (0, 0)),
            pl.BlockSpec(W_lin.shape, lambda i: (0, 0)),
            pl.BlockSpec((1, 1), lambda i: (0, 0)),
        ],
        out_specs=pl.BlockSpec((_BLOCK_ROWS, 1), lambda i: (i, 0)),
        out_shape=jax.ShapeDtypeStruct((n, 1), jnp.float32),
        compiler_params=pltpu.CompilerParams(
            dimension_semantics=("parallel",),
        ),
    )(feat, b_feat, W_feat, W_lin, b_lin2d)
